# baseline (speedup 1.0000x reference)
import functools
import math

import jax
import jax.numpy as jnp
from jax import lax
from jax.experimental import pallas as pl
from jax.experimental.pallas import tpu as pltpu
from jax.experimental.pallas import tpu_sc as plsc

_D = 1024
_NG = 128
_CAP = 64
_SUB = 32
_SH = 7
_DP = 5
_NW = 32
_CHUNK = 64


def _gelu_exact(s):
    return 0.5 * s * (1.0 + jax.lax.erf(s * (1.0 / math.sqrt(2.0))))


def _dotT(lhs, rhs, prec=None):
    return jax.lax.dot_general(
        lhs, rhs, dimension_numbers=(((1,), (1,)), ((), ())),
        preferred_element_type=jnp.float32, precision=prec)


def _tc1_kernel(x_ref, w_sh_ref, m_ref, slot_ref, counts_ref):
    i = pl.program_id(0)

    @pl.when(i == 0)
    def _init():
        counts_ref[...] = jnp.zeros_like(counts_ref)

    bt = x_ref.shape[0]
    a = _dotT(x_ref[...], w_sh_ref[...],
              prec=jax.lax.Precision.HIGHEST)
    node = jnp.zeros((bt, 1), jnp.int32)
    m = jnp.zeros((bt, _NG), jnp.float32)
    lane = jax.lax.broadcasted_iota(jnp.int32, (bt, _NG), 1)
    for _ in range(_SH):
        mask = lane == node
        s = jnp.sum(jnp.where(mask, a, 0.0), axis=1, keepdims=True)
        m = m + jnp.where(mask, _gelu_exact(s), 0.0)
        node = node * 2 + 1 + (s >= 0.0).astype(jnp.int32)
    m_ref[...] = m

    gid = node - (_NG - 1)
    oh = (lane == gid).astype(jnp.float32)
    r_i = jax.lax.broadcasted_iota(jnp.int32, (bt, bt), 0)
    c_i = jax.lax.broadcasted_iota(jnp.int32, (bt, bt), 1)
    tri = (c_i < r_i).astype(jnp.float32)
    before = jax.lax.dot_general(
        tri, oh, (((1,), (0,)), ((), ())),
        preferred_element_type=jnp.float32)
    rank = jnp.sum((before + counts_ref[0:1, :]) * oh, axis=1, keepdims=True)
    counts_ref[0:1, :] += jnp.sum(oh, axis=0, keepdims=True)
    slot = gid * _CAP + jnp.minimum(rank.astype(jnp.int32), _CAP - 1)
    eye = (r_i == c_i).astype(jnp.int32)
    slot_t = jnp.sum(slot * eye, axis=0, keepdims=True)
    slot_ref[0] = slot_t


def _tc2_kernel(xp_ref, wsub_ref, wout_ref, y_ref):
    xb = xp_ref[...]
    a = _dotT(xb, wsub_ref[0],
              prec=jax.lax.Precision.HIGHEST)
    n = xb.shape[0]
    lane = jax.lax.broadcasted_iota(jnp.int32, (n, _SUB), 1)
    j = jnp.zeros((n, 1), jnp.int32)
    m = jnp.zeros((n, _SUB), jnp.float32)
    for d in range(_DP):
        col = ((1 << d) - 1) + j
        mask = lane == col
        s = jnp.sum(jnp.where(mask, a, 0.0), axis=1, keepdims=True)
        m = m + jnp.where(mask, _gelu_exact(s), 0.0)
        j = j * 2 + (s >= 0.0).astype(jnp.int32)
    y_ref[...] = jax.lax.dot_general(
        m, wout_ref[0], (((1,), (0,)), ((), ())),
        preferred_element_type=jnp.float32)


def _tc3_kernel(m_ref, w128_ref, y_ref, out_ref):
    out_ref[...] = _dotT(m_ref[...], w128_ref[...]) + y_ref[...]


@functools.lru_cache(maxsize=None)
def _sc_kernels():
    mesh = plsc.VectorSubcoreMesh(core_axis_name="c", subcore_axis_name="s")
    scratch = [
        pltpu.VMEM((_CHUNK,), jnp.int32),
        pltpu.VMEM((_CHUNK, _D), jnp.float32),
        pltpu.SemaphoreType.DMA,
    ]

    @functools.partial(
        pl.kernel, mesh=mesh,
        out_type=jax.ShapeDtypeStruct((_NG * _CAP, _D), jnp.float32),
        scratch_types=scratch,
    )
    def scatter_k(x_hbm, slot_hbm, xpad_hbm, idx_v, rows_v, sem):
        wid = lax.axis_index("s") * 2 + lax.axis_index("c")
        row = wid // 2
        coff = (wid % 2) * 128
        for jc in range(128 // _CHUNK):
            base = wid * 128 + jc * _CHUNK
            pltpu.sync_copy(
                slot_hbm.at[row, 0, pl.ds(coff + jc * _CHUNK, _CHUNK)], idx_v)
            pltpu.sync_copy(x_hbm.at[pl.ds(base, _CHUNK)], rows_v)
            pltpu.async_copy(rows_v, xpad_hbm.at[idx_v], sem).wait()

    @functools.partial(
        pl.kernel, mesh=mesh,
        out_type=jax.ShapeDtypeStruct((_NW * 128, _D), jnp.float32),
        scratch_types=scratch,
    )
    def gather_k(ypad_hbm, slot_hbm, y_hbm, idx_v, rows_v, sem):
        wid = lax.axis_index("s") * 2 + lax.axis_index("c")
        row = wid // 2
        coff = (wid % 2) * 128
        for jc in range(128 // _CHUNK):
            base = wid * 128 + jc * _CHUNK
            pltpu.sync_copy(
                slot_hbm.at[row, 0, pl.ds(coff + jc * _CHUNK, _CHUNK)], idx_v)
            pltpu.async_copy(ypad_hbm.at[idx_v], rows_v, sem).wait()
            pltpu.sync_copy(rows_v, y_hbm.at[pl.ds(base, _CHUNK)])

    return scatter_k, gather_k


def _sc_scatter(x_flat, slot3):
    slot = slot3.reshape(-1)
    return jnp.zeros((_NG * _CAP, _D), jnp.float32).at[slot].set(x_flat)


def _sc_gather(ypad, slot3):
    return ypad[slot3.reshape(-1)]


def kernel(x, W_in, W_out):
    b, s, d = x.shape
    n_tok = b * s
    bt = 256
    x_flat = x.reshape(n_tok, d)

    m_sh, slot3 = pl.pallas_call(
        _tc1_kernel,
        grid=(n_tok // bt,),
        in_specs=[
            pl.BlockSpec((bt, d), lambda i: (i, 0)),
            pl.BlockSpec((_NG, d), lambda i: (0, 0)),
        ],
        out_specs=[
            pl.BlockSpec((bt, _NG), lambda i: (i, 0)),
            pl.BlockSpec((1, 1, bt), lambda i: (i, 0, 0)),
        ],
        out_shape=[
            jax.ShapeDtypeStruct((n_tok, _NG), jnp.float32),
            jax.ShapeDtypeStruct((n_tok // bt, 1, bt), jnp.int32),
        ],
        scratch_shapes=[pltpu.VMEM((8, _NG), jnp.float32)],
    )(x_flat, W_in)

    def subtree(tab):
        parts = [tab[(1 << (7 + dd)) - 1:(1 << (8 + dd)) - 1]
                 .reshape(_NG, 1 << dd, d) for dd in range(_DP)]
        parts.append(jnp.zeros((_NG, 1, d), tab.dtype))
        return jnp.concatenate(parts, axis=1)

    w_sub = subtree(W_in)
    wout_sub = subtree(W_out.T)

    xpad = _sc_scatter(x_flat, slot3)

    ypad = pl.pallas_call(
        _tc2_kernel,
        grid=(_NG,),
        in_specs=[
            pl.BlockSpec((_CAP, d), lambda i: (i, 0)),
            pl.BlockSpec((1, _SUB, d), lambda i: (i, 0, 0)),
            pl.BlockSpec((1, _SUB, d), lambda i: (i, 0, 0)),
        ],
        out_specs=pl.BlockSpec((_CAP, d), lambda i: (i, 0)),
        out_shape=jax.ShapeDtypeStruct((_NG * _CAP, d), jnp.float32),
    )(xpad, w_sub, wout_sub)

    y = _sc_gather(ypad, slot3)

    out = pl.pallas_call(
        _tc3_kernel,
        grid=(n_tok // bt,),
        in_specs=[
            pl.BlockSpec((bt, _NG), lambda i: (i, 0)),
            pl.BlockSpec((d, _NG), lambda i: (0, 0)),
            pl.BlockSpec((bt, d), lambda i: (i, 0)),
        ],
        out_specs=pl.BlockSpec((bt, d), lambda i: (i, 0)),
        out_shape=jax.ShapeDtypeStruct((n_tok, d), jnp.float32),
    )(m_sh, W_out, y)
    return out.reshape(b, s, d)


# speedup vs baseline: 1.1037x; 1.1037x over previous
import functools
import math

import jax
import jax.numpy as jnp
from jax import lax
from jax.experimental import pallas as pl
from jax.experimental.pallas import tpu as pltpu
from jax.experimental.pallas import tpu_sc as plsc

_D = 1024
_NG = 128
_CAP = 64
_SUB = 32
_SH = 7
_DP = 5
_NW = 32
_CHUNK = 64


def _gelu_exact(s):
    return 0.5 * s * (1.0 + jax.lax.erf(s * (1.0 / math.sqrt(2.0))))


def _dotT(lhs, rhs, prec=None):
    return jax.lax.dot_general(
        lhs, rhs, dimension_numbers=(((1,), (1,)), ((), ())),
        preferred_element_type=jnp.float32, precision=prec)


def _tc1_kernel(x_ref, w_sh_ref, m_ref, slot_ref, counts_ref):
    i = pl.program_id(0)

    @pl.when(i == 0)
    def _init():
        counts_ref[...] = jnp.zeros_like(counts_ref)

    bt = x_ref.shape[0]
    a = _dotT(x_ref[...], w_sh_ref[...],
              prec=jax.lax.Precision.HIGHEST)
    node = jnp.zeros((bt, 1), jnp.int32)
    m = jnp.zeros((bt, _NG), jnp.float32)
    lane = jax.lax.broadcasted_iota(jnp.int32, (bt, _NG), 1)
    for _ in range(_SH):
        mask = lane == node
        s = jnp.sum(jnp.where(mask, a, 0.0), axis=1, keepdims=True)
        m = m + jnp.where(mask, _gelu_exact(s), 0.0)
        node = node * 2 + 1 + (s >= 0.0).astype(jnp.int32)
    m_ref[...] = m

    gid = node - (_NG - 1)
    oh = (lane == gid).astype(jnp.float32)
    r_i = jax.lax.broadcasted_iota(jnp.int32, (bt, bt), 0)
    c_i = jax.lax.broadcasted_iota(jnp.int32, (bt, bt), 1)
    tri = (c_i < r_i).astype(jnp.float32)
    before = jax.lax.dot_general(
        tri, oh, (((1,), (0,)), ((), ())),
        preferred_element_type=jnp.float32)
    rank = jnp.sum((before + counts_ref[0:1, :]) * oh, axis=1, keepdims=True)
    counts_ref[0:1, :] += jnp.sum(oh, axis=0, keepdims=True)
    slot = gid * _CAP + jnp.minimum(rank.astype(jnp.int32), _CAP - 1)
    eye = (r_i == c_i).astype(jnp.int32)
    slot_t = jnp.sum(slot * eye, axis=0, keepdims=True)
    slot_ref[0] = slot_t


_GB = 8


def _tc2_kernel(xp_ref, wsub_ref, wout_ref, y_ref):
    lane = jax.lax.broadcasted_iota(jnp.int32, (_CAP, _SUB), 1)
    for g in range(_GB):
        xb = xp_ref[g * _CAP:(g + 1) * _CAP, :]
        a = _dotT(xb, wsub_ref[g],
                  prec=jax.lax.Precision.HIGHEST)
        j = jnp.zeros((_CAP, 1), jnp.int32)
        m = jnp.zeros((_CAP, _SUB), jnp.float32)
        for d in range(_DP):
            col = ((1 << d) - 1) + j
            mask = lane == col
            s = jnp.sum(jnp.where(mask, a, 0.0), axis=1, keepdims=True)
            m = m + jnp.where(mask, _gelu_exact(s), 0.0)
            j = j * 2 + (s >= 0.0).astype(jnp.int32)
        y_ref[g * _CAP:(g + 1) * _CAP, :] = jax.lax.dot_general(
            m, wout_ref[g], (((1,), (0,)), ((), ())),
            preferred_element_type=jnp.float32)


def _tc3_kernel(m_ref, w128_ref, y_ref, out_ref):
    out_ref[...] = _dotT(m_ref[...], w128_ref[...]) + y_ref[...]


@functools.lru_cache(maxsize=None)
def _sc_kernels():
    mesh = plsc.VectorSubcoreMesh(core_axis_name="c", subcore_axis_name="s")
    scratch = [
        pltpu.VMEM((_CHUNK,), jnp.int32),
        pltpu.VMEM((_CHUNK, _D), jnp.float32),
        pltpu.SemaphoreType.DMA,
    ]

    @functools.partial(
        pl.kernel, mesh=mesh,
        out_type=jax.ShapeDtypeStruct((_NG * _CAP, _D), jnp.float32),
        scratch_types=scratch,
    )
    def scatter_k(x_hbm, slot_hbm, xpad_hbm, idx_v, rows_v, sem):
        wid = lax.axis_index("s") * 2 + lax.axis_index("c")
        row = wid // 2
        coff = (wid % 2) * 128
        for jc in range(128 // _CHUNK):
            base = wid * 128 + jc * _CHUNK
            pltpu.sync_copy(
                slot_hbm.at[row, 0, pl.ds(coff + jc * _CHUNK, _CHUNK)], idx_v)
            pltpu.sync_copy(x_hbm.at[pl.ds(base, _CHUNK)], rows_v)
            pltpu.async_copy(rows_v, xpad_hbm.at[idx_v], sem).wait()

    @functools.partial(
        pl.kernel, mesh=mesh,
        out_type=jax.ShapeDtypeStruct((_NW * 128, _D), jnp.float32),
        scratch_types=scratch,
    )
    def gather_k(ypad_hbm, slot_hbm, y_hbm, idx_v, rows_v, sem):
        wid = lax.axis_index("s") * 2 + lax.axis_index("c")
        row = wid // 2
        coff = (wid % 2) * 128
        for jc in range(128 // _CHUNK):
            base = wid * 128 + jc * _CHUNK
            pltpu.sync_copy(
                slot_hbm.at[row, 0, pl.ds(coff + jc * _CHUNK, _CHUNK)], idx_v)
            pltpu.async_copy(ypad_hbm.at[idx_v], rows_v, sem).wait()
            pltpu.sync_copy(rows_v, y_hbm.at[pl.ds(base, _CHUNK)])

    return scatter_k, gather_k


def _sc_scatter(x_flat, slot3):
    return _sc_kernels()[0](x_flat, slot3)


def _sc_gather(ypad, slot3):
    return _sc_kernels()[1](ypad, slot3)


def kernel(x, W_in, W_out):
    b, s, d = x.shape
    n_tok = b * s
    bt = 256
    x_flat = x.reshape(n_tok, d)

    m_sh, slot3 = pl.pallas_call(
        _tc1_kernel,
        grid=(n_tok // bt,),
        in_specs=[
            pl.BlockSpec((bt, d), lambda i: (i, 0)),
            pl.BlockSpec((_NG, d), lambda i: (0, 0)),
        ],
        out_specs=[
            pl.BlockSpec((bt, _NG), lambda i: (i, 0)),
            pl.BlockSpec((1, 1, bt), lambda i: (i, 0, 0)),
        ],
        out_shape=[
            jax.ShapeDtypeStruct((n_tok, _NG), jnp.float32),
            jax.ShapeDtypeStruct((n_tok // bt, 1, bt), jnp.int32),
        ],
        scratch_shapes=[pltpu.VMEM((8, _NG), jnp.float32)],
    )(x_flat, W_in)

    def subtree(tab):
        parts = [tab[(1 << (7 + dd)) - 1:(1 << (8 + dd)) - 1]
                 .reshape(_NG, 1 << dd, d) for dd in range(_DP)]
        parts.append(jnp.zeros((_NG, 1, d), tab.dtype))
        return jnp.concatenate(parts, axis=1)

    w_sub = subtree(W_in)
    wout_sub = subtree(W_out.T)

    xpad = _sc_scatter(x_flat, slot3)

    ypad = pl.pallas_call(
        _tc2_kernel,
        grid=(_NG // _GB,),
        in_specs=[
            pl.BlockSpec((_GB * _CAP, d), lambda i: (i, 0)),
            pl.BlockSpec((_GB, _SUB, d), lambda i: (i, 0, 0)),
            pl.BlockSpec((_GB, _SUB, d), lambda i: (i, 0, 0)),
        ],
        out_specs=pl.BlockSpec((_GB * _CAP, d), lambda i: (i, 0)),
        out_shape=jax.ShapeDtypeStruct((_NG * _CAP, d), jnp.float32),
    )(xpad, w_sub, wout_sub)

    y = _sc_gather(ypad, slot3)

    out = pl.pallas_call(
        _tc3_kernel,
        grid=(n_tok // bt,),
        in_specs=[
            pl.BlockSpec((bt, _NG), lambda i: (i, 0)),
            pl.BlockSpec((d, _NG), lambda i: (0, 0)),
            pl.BlockSpec((bt, d), lambda i: (i, 0)),
        ],
        out_specs=pl.BlockSpec((bt, d), lambda i: (i, 0)),
        out_shape=jax.ShapeDtypeStruct((n_tok, d), jnp.float32),
    )(m_sh, W_out, y)
    return out.reshape(b, s, d)


# speedup vs baseline: 1.5469x; 1.4016x over previous
import functools
import math

import jax
import jax.numpy as jnp
from jax import lax
from jax.experimental import pallas as pl
from jax.experimental.pallas import tpu as pltpu
from jax.experimental.pallas import tpu_sc as plsc

_D = 1024
_NG = 128
_CAP = 64
_SUB = 32
_SH = 7
_DP = 5
_NW = 32
_CHUNK = 64


def _gelu_exact(s):
    return 0.5 * s * (1.0 + jax.lax.erf(s * (1.0 / math.sqrt(2.0))))


def _dotT(lhs, rhs, prec=None):
    return jax.lax.dot_general(
        lhs, rhs, dimension_numbers=(((1,), (1,)), ((), ())),
        preferred_element_type=jnp.float32, precision=prec)


def _tc1_kernel(x_ref, w_sh_ref, m_ref, slot_ref, counts_ref):
    i = pl.program_id(0)

    @pl.when(i == 0)
    def _init():
        counts_ref[...] = jnp.zeros_like(counts_ref)

    bt = x_ref.shape[0]
    a = _dotT(x_ref[...], w_sh_ref[...],
              prec=jax.lax.Precision.HIGHEST)
    node = jnp.zeros((bt, 1), jnp.int32)
    m = jnp.zeros((bt, _NG), jnp.float32)
    lane = jax.lax.broadcasted_iota(jnp.int32, (bt, _NG), 1)
    for _ in range(_SH):
        mask = lane == node
        s = jnp.sum(jnp.where(mask, a, 0.0), axis=1, keepdims=True)
        m = m + jnp.where(mask, _gelu_exact(s), 0.0)
        node = node * 2 + 1 + (s >= 0.0).astype(jnp.int32)
    m_ref[...] = m

    gid = node - (_NG - 1)
    oh = (lane == gid).astype(jnp.float32)
    r_i = jax.lax.broadcasted_iota(jnp.int32, (bt, bt), 0)
    c_i = jax.lax.broadcasted_iota(jnp.int32, (bt, bt), 1)
    tri = (c_i < r_i).astype(jnp.float32)
    before = jax.lax.dot_general(
        tri, oh, (((1,), (0,)), ((), ())),
        preferred_element_type=jnp.float32)
    rank = jnp.sum((before + counts_ref[0:1, :]) * oh, axis=1, keepdims=True)
    counts_ref[0:1, :] += jnp.sum(oh, axis=0, keepdims=True)
    slot = gid * _CAP + jnp.minimum(rank.astype(jnp.int32), _CAP - 1)
    eye = (r_i == c_i).astype(jnp.int32)
    slot_t = jnp.sum(slot * eye, axis=0, keepdims=True)
    slot_ref[0] = slot_t


_GB = 8


def _tc2_kernel(xp_ref, wsub_ref, wout_ref, y_ref):
    lane = jax.lax.broadcasted_iota(jnp.int32, (_CAP, _SUB), 1)
    for g in range(_GB):
        xb = xp_ref[g * _CAP:(g + 1) * _CAP, :]
        a = _dotT(xb, wsub_ref[g],
                  prec=jax.lax.Precision.HIGHEST)
        j = jnp.zeros((_CAP, 1), jnp.int32)
        m = jnp.zeros((_CAP, _SUB), jnp.float32)
        for d in range(_DP):
            col = ((1 << d) - 1) + j
            mask = lane == col
            s = jnp.sum(jnp.where(mask, a, 0.0), axis=1, keepdims=True)
            m = m + jnp.where(mask, _gelu_exact(s), 0.0)
            j = j * 2 + (s >= 0.0).astype(jnp.int32)
        y_ref[g * _CAP:(g + 1) * _CAP, :] = jax.lax.dot_general(
            m, wout_ref[g], (((1,), (0,)), ((), ())),
            preferred_element_type=jnp.float32)


def _tc3_kernel(m_ref, w128_ref, y_ref, out_ref):
    out_ref[...] = _dotT(m_ref[...], w128_ref[...]) + y_ref[...]


@functools.lru_cache(maxsize=None)
def _sc_kernels():
    mesh = plsc.VectorSubcoreMesh(core_axis_name="c", subcore_axis_name="s")
    scratch = [
        pltpu.VMEM((_CHUNK,), jnp.int32),
        pltpu.VMEM((_CHUNK, _D), jnp.float32),
        pltpu.SemaphoreType.DMA,
    ]

    @functools.partial(
        pl.kernel, mesh=mesh,
        out_type=jax.ShapeDtypeStruct((_NG * _CAP, _D), jnp.float32),
        scratch_types=scratch,
    )
    def scatter_k(x_hbm, slot_hbm, xpad_hbm, idx_v, rows_v, sem):
        wid = lax.axis_index("s") * 2 + lax.axis_index("c")
        row = wid // 2
        coff = (wid % 2) * 128
        for jc in range(128 // _CHUNK):
            base = wid * 128 + jc * _CHUNK
            pltpu.sync_copy(
                slot_hbm.at[row, 0, pl.ds(coff + jc * _CHUNK, _CHUNK)], idx_v)
            pltpu.sync_copy(x_hbm.at[pl.ds(base, _CHUNK)], rows_v)
            pltpu.async_copy(rows_v, xpad_hbm.at[idx_v], sem).wait()

    @functools.partial(
        pl.kernel, mesh=mesh,
        out_type=jax.ShapeDtypeStruct((_NW * 128, _D), jnp.float32),
        scratch_types=scratch,
    )
    def gather_k(ypad_hbm, slot_hbm, y_hbm, idx_v, rows_v, sem):
        wid = lax.axis_index("s") * 2 + lax.axis_index("c")
        row = wid // 2
        coff = (wid % 2) * 128
        for jc in range(128 // _CHUNK):
            base = wid * 128 + jc * _CHUNK
            pltpu.sync_copy(
                slot_hbm.at[row, 0, pl.ds(coff + jc * _CHUNK, _CHUNK)], idx_v)
            pltpu.async_copy(ypad_hbm.at[idx_v], rows_v, sem).wait()
            pltpu.sync_copy(rows_v, y_hbm.at[pl.ds(base, _CHUNK)])

    return scatter_k, gather_k


def _sc_scatter(x_flat, slot3):
    return _sc_kernels()[0](x_flat, slot3)


def _sc_gather(ypad, slot3):
    return _sc_kernels()[1](ypad, slot3)


def kernel(x, W_in, W_out):
    _BISECT = 'B'
    b, s, d = x.shape
    n_tok = b * s
    bt = 256
    x_flat = x.reshape(n_tok, d)

    m_sh, slot3 = pl.pallas_call(
        _tc1_kernel,
        grid=(n_tok // bt,),
        in_specs=[
            pl.BlockSpec((bt, d), lambda i: (i, 0)),
            pl.BlockSpec((_NG, d), lambda i: (0, 0)),
        ],
        out_specs=[
            pl.BlockSpec((bt, _NG), lambda i: (i, 0)),
            pl.BlockSpec((1, 1, bt), lambda i: (i, 0, 0)),
        ],
        out_shape=[
            jax.ShapeDtypeStruct((n_tok, _NG), jnp.float32),
            jax.ShapeDtypeStruct((n_tok // bt, 1, bt), jnp.int32),
        ],
        scratch_shapes=[pltpu.VMEM((8, _NG), jnp.float32)],
    )(x_flat, W_in)

    def subtree(tab):
        parts = [tab[(1 << (7 + dd)) - 1:(1 << (8 + dd)) - 1]
                 .reshape(_NG, 1 << dd, d) for dd in range(_DP)]
        parts.append(jnp.zeros((_NG, 1, d), tab.dtype))
        return jnp.concatenate(parts, axis=1)

    w_sub = subtree(W_in)
    wout_sub = subtree(W_in) if _BISECT == 'B' else subtree(W_out.T)

    xpad = _sc_scatter(x_flat, slot3)

    ypad = pl.pallas_call(
        _tc2_kernel,
        grid=(_NG // _GB,),
        in_specs=[
            pl.BlockSpec((_GB * _CAP, d), lambda i: (i, 0)),
            pl.BlockSpec((_GB, _SUB, d), lambda i: (i, 0, 0)),
            pl.BlockSpec((_GB, _SUB, d), lambda i: (i, 0, 0)),
        ],
        out_specs=pl.BlockSpec((_GB * _CAP, d), lambda i: (i, 0)),
        out_shape=jax.ShapeDtypeStruct((_NG * _CAP, d), jnp.float32),
    )(xpad, w_sub, wout_sub)

    y = x_flat if _BISECT == 'A' else _sc_gather(ypad, slot3)

    out = pl.pallas_call(
        _tc3_kernel,
        grid=(n_tok // bt,),
        in_specs=[
            pl.BlockSpec((bt, _NG), lambda i: (i, 0)),
            pl.BlockSpec((d, _NG), lambda i: (0, 0)),
            pl.BlockSpec((bt, d), lambda i: (i, 0)),
        ],
        out_specs=pl.BlockSpec((bt, d), lambda i: (i, 0)),
        out_shape=jax.ShapeDtypeStruct((n_tok, d), jnp.float32),
    )(m_sh, W_out, y)
    return out.reshape(b, s, d)


# speedup vs baseline: 2.4028x; 1.5533x over previous
import functools
import math

import jax
import jax.numpy as jnp
from jax.experimental import pallas as pl
from jax.experimental.pallas import tpu as pltpu

_DEPTH = 11
_LEVELS = _DEPTH + 1
_NN = 2 ** (_DEPTH + 1) - 1
_LANE = 128


def _gelu_exact(s):
    return 0.5 * s * (1.0 + jax.lax.erf(s * (1.0 / math.sqrt(2.0))))


def _dotT(lhs, rhs):
    return jax.lax.dot_general(
        lhs, rhs,
        dimension_numbers=(((1,), (1,)), ((), ())),
        preferred_element_type=jnp.float32,
    )


_SAFE_COLS = 512


def _fff_block_kernel(x_ref, w_in_hi_ref, w_in_lo_ref, w_sh_ref, w_out_ref,
                      out_ref, m_ref, a_ref):
    bt = x_ref.shape[0]
    x = x_ref[...]
    x_hi = x.astype(jnp.bfloat16)
    x_lo = (x - x_hi.astype(jnp.float32)).astype(jnp.bfloat16)
    w_hi = w_in_hi_ref[...]
    w_lo = w_in_lo_ref[...]
    a_ref[...] = _dotT(x_hi, w_hi) + _dotT(x_hi, w_lo) + _dotT(x_lo, w_hi)
    a_ref[:, 0:_SAFE_COLS] = jax.lax.dot_general(
        x, w_sh_ref[...],
        dimension_numbers=(((1,), (1,)), ((), ())),
        preferred_element_type=jnp.float32,
        precision=jax.lax.Precision.HIGHEST,
    )
    m_ref[...] = jnp.zeros_like(m_ref)
    node = jnp.zeros((bt, 1), jnp.int32)
    for lvl in range(_LEVELS):
        lo = (1 << lvl) - 1
        width = 1 << lvl
        c0 = (lo // _LANE) * _LANE
        c1 = min(_NN, ((lo + width + _LANE - 1) // _LANE) * _LANE)
        iota = jax.lax.broadcasted_iota(jnp.int32, (bt, c1 - c0), 1) + c0
        mask = iota == node
        a_sl = a_ref[:, c0:c1]
        s = jnp.sum(jnp.where(mask, a_sl, 0.0), axis=1, keepdims=True)
        g = _gelu_exact(s)
        m_ref[:, c0:c1] += jnp.where(mask, g, 0.0)
        node = node * 2 + 1 + (s >= 0.0).astype(jnp.int32)
    out_ref[...] = jax.lax.dot_general(
        m_ref[...], w_out_ref[...],
        dimension_numbers=(((1,), (1,)), ((), ())),
        preferred_element_type=jnp.float32,
    )


def kernel(x, W_in, W_out):
    b, s, d = x.shape
    n_tok = b * s
    bt = 256
    grid = n_tok // bt
    x_flat = x.reshape(n_tok, d)
    w_in_hi = W_in.astype(jnp.bfloat16)
    w_in_lo = (W_in - jax.lax.optimization_barrier(w_in_hi)
               .astype(jnp.float32)).astype(jnp.bfloat16)
    out = pl.pallas_call(
        _fff_block_kernel,
        grid=(grid,),
        in_specs=[
            pl.BlockSpec((bt, d), lambda i: (i, 0)),
            pl.BlockSpec((_NN, d), lambda i: (0, 0)),
            pl.BlockSpec((_NN, d), lambda i: (0, 0)),
            pl.BlockSpec((_SAFE_COLS, d), lambda i: (0, 0)),
            pl.BlockSpec((d, _NN), lambda i: (0, 0)),
        ],
        out_specs=pl.BlockSpec((bt, d), lambda i: (i, 0)),
        out_shape=jax.ShapeDtypeStruct((n_tok, d), jnp.float32),
        scratch_shapes=[pltpu.VMEM((bt, _NN), jnp.float32),
                        pltpu.VMEM((bt, _NN), jnp.float32)],
    )(x_flat, w_in_hi, w_in_lo, W_in, W_out)
    return out.reshape(b, s, d)


# speedup vs baseline: 2.5349x; 1.0550x over previous
import functools
import math

import jax
import jax.numpy as jnp
from jax.experimental import pallas as pl
from jax.experimental.pallas import tpu as pltpu

_DEPTH = 11
_LEVELS = _DEPTH + 1
_NN = 2 ** (_DEPTH + 1) - 1
_LANE = 128


def _gelu_exact(s):
    return 0.5 * s * (1.0 + jax.lax.erf(s * (1.0 / math.sqrt(2.0))))


def _dotT(lhs, rhs):
    return jax.lax.dot_general(
        lhs, rhs,
        dimension_numbers=(((1,), (1,)), ((), ())),
        preferred_element_type=jnp.float32,
    )


_SAFE_COLS = 512


def _fff_block_kernel(x_ref, w_in_hi_ref, w_in_lo_ref, w_sh_ref, w_out_ref,
                      out_ref, m_ref, a_ref):
    bt = x_ref.shape[0]
    x = x_ref[...]
    x_hi = x.astype(jnp.bfloat16)
    x_lo = (x - x_hi.astype(jnp.float32)).astype(jnp.bfloat16)
    w_hi = w_in_hi_ref[...]
    w_lo = w_in_lo_ref[...]
    a_ref[:, _SAFE_COLS:] = (_dotT(x_hi, w_hi) + _dotT(x_hi, w_lo)
                             + _dotT(x_lo, w_hi))
    a_ref[:, 0:_SAFE_COLS] = jax.lax.dot_general(
        x, w_sh_ref[...],
        dimension_numbers=(((1,), (1,)), ((), ())),
        preferred_element_type=jnp.float32,
        precision=jax.lax.Precision.HIGHEST,
    )
    m_ref[...] = jnp.zeros_like(m_ref)
    node = jnp.zeros((bt, 1), jnp.int32)
    for lvl in range(_LEVELS):
        lo = (1 << lvl) - 1
        width = 1 << lvl
        c0 = (lo // _LANE) * _LANE
        c1 = min(_NN, ((lo + width + _LANE - 1) // _LANE) * _LANE)
        iota = jax.lax.broadcasted_iota(jnp.int32, (bt, c1 - c0), 1) + c0
        mask = iota == node
        a_sl = a_ref[:, c0:c1]
        s = jnp.sum(jnp.where(mask, a_sl, 0.0), axis=1, keepdims=True)
        g = _gelu_exact(s)
        m_ref[:, c0:c1] += jnp.where(mask, g, 0.0)
        node = node * 2 + 1 + (s >= 0.0).astype(jnp.int32)
    out_ref[...] = jax.lax.dot_general(
        m_ref[...], w_out_ref[...],
        dimension_numbers=(((1,), (1,)), ((), ())),
        preferred_element_type=jnp.float32,
    )


def kernel(x, W_in, W_out):
    b, s, d = x.shape
    n_tok = b * s
    bt = 256
    grid = n_tok // bt
    x_flat = x.reshape(n_tok, d)
    w_in_hi = W_in.astype(jnp.bfloat16)
    w_in_lo = (W_in - jax.lax.optimization_barrier(w_in_hi)
               .astype(jnp.float32)).astype(jnp.bfloat16)
    deep = _NN - _SAFE_COLS
    out = pl.pallas_call(
        _fff_block_kernel,
        grid=(grid,),
        in_specs=[
            pl.BlockSpec((bt, d), lambda i: (i, 0)),
            pl.BlockSpec((deep, d), lambda i: (0, 0)),
            pl.BlockSpec((deep, d), lambda i: (0, 0)),
            pl.BlockSpec((_SAFE_COLS, d), lambda i: (0, 0)),
            pl.BlockSpec((d, _NN), lambda i: (0, 0)),
        ],
        out_specs=pl.BlockSpec((bt, d), lambda i: (i, 0)),
        out_shape=jax.ShapeDtypeStruct((n_tok, d), jnp.float32),
        scratch_shapes=[pltpu.VMEM((bt, _NN), jnp.float32),
                        pltpu.VMEM((bt, _NN), jnp.float32)],
    )(x_flat, w_in_hi[_SAFE_COLS:], w_in_lo[_SAFE_COLS:], W_in, W_out)
    return out.reshape(b, s, d)
